# baseline (speedup 1.0000x reference)
import functools

import jax
import jax.numpy as jnp
from jax import lax
from jax.experimental import pallas as pl
from jax.experimental.pallas import tpu as pltpu
from jax.experimental.pallas import tpu_sc as plsc

_D = 256
_S = 8192
_B = 4
_B_SC = 2
_B_TC = _B - _B_SC
_NC = 2
_NS = 16
_NW = _NC * _NS
_ROWS = _S // _NW
_CHUNK = 128

_mesh = plsc.VectorSubcoreMesh(core_axis_name="c", subcore_axis_name="s")


@functools.partial(
    pl.kernel,
    mesh=_mesh,
    out_type=jax.ShapeDtypeStruct((_B_SC, _S, _D), jnp.float32),
    scratch_types=[
        pltpu.VMEM((_ROWS, _D), jnp.float32),
        pltpu.SemaphoreType.DMA,
        pltpu.SemaphoreType.DMA,
    ],
)
def _broadcast_rows_sc(emb_hbm, out_hbm, buf, rsem, wsem):
    wid = lax.axis_index("s") * _NC + lax.axis_index("c")
    base = wid * _ROWS
    nchunks = _ROWS // _CHUNK
    reads = [
        pltpu.async_copy(
            emb_hbm.at[pl.ds(base + i * _CHUNK, _CHUNK)],
            buf.at[pl.ds(i * _CHUNK, _CHUNK)],
            rsem,
        )
        for i in range(nchunks)
    ]
    writes = []
    for i in range(nchunks):
        reads[i].wait()
        writes += [
            pltpu.async_copy(
                buf.at[pl.ds(i * _CHUNK, _CHUNK)],
                out_hbm.at[b, pl.ds(base + i * _CHUNK, _CHUNK)],
                wsem,
            )
            for b in range(_B_SC)
        ]
    for w in writes:
        w.wait()


_TC_CHUNK = 512


def _broadcast_tc_body(emb_ref, out_ref):
    out_ref[...] = jnp.broadcast_to(
        emb_ref[...][None], (_B_TC, _TC_CHUNK, _D)
    )


_broadcast_rows_tc = pl.pallas_call(
    _broadcast_tc_body,
    grid=(_S // _TC_CHUNK,),
    in_specs=[pl.BlockSpec((_TC_CHUNK, _D), lambda i: (i, 0))],
    out_specs=pl.BlockSpec((_B_TC, _TC_CHUNK, _D), lambda i: (0, i, 0)),
    out_shape=jax.ShapeDtypeStruct((_B_TC, _S, _D), jnp.float32),
)


def kernel(seq_in_embeds, embedding):
    del seq_in_embeds
    sc_half = _broadcast_rows_sc(embedding)
    tc_half = _broadcast_rows_tc(embedding)
    return jnp.concatenate([sc_half, tc_half], axis=0)


# speedup vs baseline: 1.8177x; 1.8177x over previous
import functools

import jax
import jax.numpy as jnp
from jax import lax
from jax.experimental import pallas as pl
from jax.experimental.pallas import tpu as pltpu
from jax.experimental.pallas import tpu_sc as plsc

_D = 256
_S = 8192
_B = 4
_NC = 2
_NS = 16
_NW = _NC * _NS
_ROWS = _S // _NW
_CHUNKS = (32, 96, 128)

_mesh = plsc.VectorSubcoreMesh(core_axis_name="c", subcore_axis_name="s")


@functools.partial(
    pl.kernel,
    mesh=_mesh,
    out_type=jax.ShapeDtypeStruct((_B, _S, _D), jnp.float32),
    scratch_types=[
        pltpu.VMEM((_ROWS, _D), jnp.float32),
        pltpu.SemaphoreType.DMA,
        pltpu.SemaphoreType.DMA,
    ],
)
def _broadcast_rows(emb_hbm, out_hbm, buf, rsem, wsem):
    wid = lax.axis_index("s") * _NC + lax.axis_index("c")
    base = wid * _ROWS
    offs = [sum(_CHUNKS[:i]) for i in range(len(_CHUNKS))]
    reads = [
        pltpu.async_copy(
            emb_hbm.at[pl.ds(base + o, c)],
            buf.at[pl.ds(o, c)],
            rsem,
        )
        for o, c in zip(offs, _CHUNKS)
    ]
    writes = []
    for i, (o, c) in enumerate(zip(offs, _CHUNKS)):
        reads[i].wait()
        writes += [
            pltpu.async_copy(
                buf.at[pl.ds(o, c)],
                out_hbm.at[b, pl.ds(base + o, c)],
                wsem,
            )
            for b in range(_B)
        ]
    for w in writes:
        w.wait()


def kernel(seq_in_embeds, embedding):
    del seq_in_embeds
    return _broadcast_rows(embedding)


# speedup vs baseline: 1.8324x; 1.0081x over previous
import functools

import jax
import jax.numpy as jnp
from jax import lax
from jax.experimental import pallas as pl
from jax.experimental.pallas import tpu as pltpu
from jax.experimental.pallas import tpu_sc as plsc

_D = 256
_S = 8192
_B = 4
_NC = 2
_NS = 16
_NW = _NC * _NS
_ROWS = _S // _NW
_CHUNK = 128

_mesh = plsc.VectorSubcoreMesh(core_axis_name="c", subcore_axis_name="s")


@functools.partial(
    pl.kernel,
    mesh=_mesh,
    out_type=jax.ShapeDtypeStruct((_B, _S, _D), jnp.float32),
    scratch_types=[
        pltpu.VMEM((_ROWS, _D), jnp.float32),
        pltpu.SemaphoreType.DMA,
        pltpu.SemaphoreType.DMA,
    ],
)
def _broadcast_rows(emb_hbm, out_hbm, buf, rsem, wsem):
    wid = lax.axis_index("s") * _NC + lax.axis_index("c")
    base = wid * _ROWS
    nchunks = _ROWS // _CHUNK
    reads = [
        pltpu.async_copy(
            emb_hbm.at[pl.ds(base + i * _CHUNK, _CHUNK)],
            buf.at[pl.ds(i * _CHUNK, _CHUNK)],
            rsem,
        )
        for i in range(nchunks)
    ]
    writes = []
    for i in range(nchunks):
        reads[i].wait()
        writes += [
            pltpu.async_copy(
                buf.at[pl.ds(i * _CHUNK, _CHUNK)],
                out_hbm.at[(wid + j) % _B, pl.ds(base + i * _CHUNK, _CHUNK)],
                wsem,
            )
            for j in range(_B)
        ]
    for w in writes:
        w.wait()


def kernel(seq_in_embeds, embedding):
    del seq_in_embeds
    return _broadcast_rows(embedding)
